# baseline (speedup 1.0000x reference)
import jax
import jax.numpy as jnp
from jax.experimental import pallas as pl
from jax.experimental.pallas import tpu as pltpu

_N = 8192
_D = 16
_K = 10
_R = 128
_W = 128
_C = _N // _W

_SORT10 = [
    (0, 5), (1, 6), (2, 7), (3, 8), (4, 9),
    (0, 3), (1, 4), (5, 8), (6, 9),
    (0, 2), (3, 6), (7, 9),
    (0, 1), (2, 4), (5, 7), (8, 9),
    (1, 2), (3, 5), (4, 6), (7, 8),
    (1, 3), (2, 5), (4, 7), (6, 8),
    (2, 3), (4, 5), (6, 7),
    (3, 4), (5, 6),
]
_SORT4 = [(0, 2), (1, 3), (0, 1), (2, 3), (1, 2)]
_CLEAN = [
    (0, 8), (1, 9), (2, 6), (3, 7), (4, 8), (5, 9),
    (2, 4), (3, 5), (6, 8), (7, 9),
    (0, 1), (2, 3), (4, 5), (6, 7), (8, 9),
]


def _apply_net(vs, net):
    vs = list(vs)
    for i, j in net:
        hi = jnp.maximum(vs[i], vs[j])
        lo = jnp.minimum(vs[i], vs[j])
        vs[i], vs[j] = hi, lo
    return vs


def _merge10(a, b, clean):
    c = [jnp.maximum(a[i], b[_K - 1 - i]) for i in range(_K)]
    if clean:
        c = _apply_net(c, _CLEAN)
    return c


def _tc_body(e_blk_ref, et_ref, out_ref):
    a = jnp.dot(e_blk_ref[...], et_ref[...], preferred_element_type=jnp.float32)

    groups = []
    for g in range(6):
        chunks = [a[:, (10 * g + c) * _W:(10 * g + c + 1) * _W] for c in range(_K)]
        groups.append(_apply_net(chunks, _SORT10))
    rest = [a[:, (60 + c) * _W:(61 + c) * _W] for c in range(4)]
    rest = _apply_net(rest, _SORT4)
    ninf = jnp.full((_R, _W), -jnp.inf, dtype=jnp.float32)
    groups.append(rest + [ninf] * 6)
    m01 = _merge10(groups[0], groups[1], True)
    m23 = _merge10(groups[2], groups[3], True)
    m45 = _merge10(groups[4], groups[5], True)
    ma = _merge10(m01, m23, True)
    mb = _merge10(m45, groups[6], True)
    lanes = _merge10(ma, mb, True)

    cur = lanes
    width = _W
    for level in range(3):
        half = width // 2
        av = [x[:, :half] for x in cur]
        bv = [x[:, half:] for x in cur]
        cur = _merge10(av, bv, clean=(level < 2))
        width = half
    cand = jnp.concatenate(cur, axis=1)
    colf = jax.lax.broadcasted_iota(jnp.int32, cand.shape, 1).astype(jnp.float32)
    work = cand
    m0r = None
    tr = None
    for it in range(_K):
        m = jnp.max(work, axis=1, keepdims=True)
        if it == 0:
            m0r = m
        cv = jnp.where(work == m, colf, jnp.float32(1e9))
        idx = jnp.min(cv, axis=1, keepdims=True)
        work = jnp.where(cv == idx, -jnp.inf, work)
        tr = m
    t = jnp.maximum(tr, 0.0)
    m0 = jnp.maximum(m0r, 0.0)

    ti = jax.lax.bitcast_convert_type(t, jnp.int32)
    t_lo = jax.lax.bitcast_convert_type(ti - 1, jnp.float32)
    thr = jnp.where(t > 0.0, t_lo, 0.0)
    selb = a > thr
    em = jnp.exp(-m0)
    p = jnp.where(selb, jnp.exp(a - m0), em)
    z = jnp.sum(p, axis=1, keepdims=True)
    out_ref[...] = p * (1.0 / z)

    cnt = jnp.zeros((_R, 1), dtype=jnp.float32)
    for j in range(_K):
        cnt = cnt + jnp.sum(
            jnp.where(lanes[j] >= t, 1.0, 0.0), axis=1, keepdims=True
        )
    sat = jnp.max(
        jnp.where(lanes[_K - 1] == t, 1.0, 0.0), axis=1, keepdims=True
    )
    bad = jnp.where(t > 0.0, cnt + 8192.0 * sat, 0.0)
    anomaly = jnp.max(bad) > 10.0

    @pl.when(anomaly)
    def _exact_topk():
        ar = jnp.maximum(a, 0.0)
        colf2 = jax.lax.broadcasted_iota(jnp.int32, ar.shape, 1).astype(jnp.float32)
        wk = ar
        for _ in range(_K):
            mm = jnp.max(wk, axis=1, keepdims=True)
            cv2 = jnp.where(wk == mm, colf2, jnp.float32(1e9))
            ix = jnp.min(cv2, axis=1, keepdims=True)
            wk = jnp.where(cv2 == ix, jnp.float32(-1.0), wk)
        p2 = jnp.where(wk < 0.0, jnp.exp(ar - m0), em)
        z2 = jnp.sum(p2, axis=1, keepdims=True)
        out_ref[...] = p2 * (1.0 / z2)


@jax.jit
def kernel(node_emb):
    et = node_emb.T
    return pl.pallas_call(
        _tc_body,
        grid=(_N // _R,),
        in_specs=[
            pl.BlockSpec((_R, _D), lambda i: (i, 0)),
            pl.BlockSpec((_D, _N), lambda i: (0, 0)),
        ],
        out_specs=pl.BlockSpec((_R, _N), lambda i: (i, 0)),
        out_shape=jax.ShapeDtypeStruct((_N, _N), jnp.float32),
        compiler_params=pltpu.CompilerParams(
            dimension_semantics=("arbitrary",)
        ),
    )(node_emb, et)


# speedup vs baseline: 1.2001x; 1.2001x over previous
import jax
import jax.numpy as jnp
from jax.experimental import pallas as pl
from jax.experimental.pallas import tpu as pltpu

_N = 8192
_D = 16
_K = 10
_R = 256
_W = 128
_C = _N // _W

_SORT10 = [
    (0, 5), (1, 6), (2, 7), (3, 8), (4, 9),
    (0, 3), (1, 4), (5, 8), (6, 9),
    (0, 2), (3, 6), (7, 9),
    (0, 1), (2, 4), (5, 7), (8, 9),
    (1, 2), (3, 5), (4, 6), (7, 8),
    (1, 3), (2, 5), (4, 7), (6, 8),
    (2, 3), (4, 5), (6, 7),
    (3, 4), (5, 6),
]
_SORT4 = [(0, 2), (1, 3), (0, 1), (2, 3), (1, 2)]
_CLEAN = [
    (0, 8), (1, 9), (2, 6), (3, 7), (4, 8), (5, 9),
    (2, 4), (3, 5), (6, 8), (7, 9),
    (0, 1), (2, 3), (4, 5), (6, 7), (8, 9),
]


def _apply_net(vs, net):
    vs = list(vs)
    for i, j in net:
        hi = jnp.maximum(vs[i], vs[j])
        lo = jnp.minimum(vs[i], vs[j])
        vs[i], vs[j] = hi, lo
    return vs


def _merge10(a, b, clean):
    c = [jnp.maximum(a[i], b[_K - 1 - i]) for i in range(_K)]
    if clean:
        c = _apply_net(c, _CLEAN)
    return c


def _tc_body(e_blk_ref, et_ref, out_ref):
    a = jnp.dot(e_blk_ref[...], et_ref[...], preferred_element_type=jnp.float32)

    groups = []
    for g in range(6):
        chunks = [a[:, (10 * g + c) * _W:(10 * g + c + 1) * _W] for c in range(_K)]
        groups.append(_apply_net(chunks, _SORT10))
    rest = [a[:, (60 + c) * _W:(61 + c) * _W] for c in range(4)]
    rest = _apply_net(rest, _SORT4)
    ninf = jnp.full((_R, _W), -jnp.inf, dtype=jnp.float32)
    groups.append(rest + [ninf] * 6)
    m01 = _merge10(groups[0], groups[1], True)
    m23 = _merge10(groups[2], groups[3], True)
    m45 = _merge10(groups[4], groups[5], True)
    ma = _merge10(m01, m23, True)
    mb = _merge10(m45, groups[6], True)
    lanes = _merge10(ma, mb, True)

    cur = lanes
    width = _W
    for level in range(3):
        half = width // 2
        av = [x[:, :half] for x in cur]
        bv = [x[:, half:] for x in cur]
        cur = _merge10(av, bv, clean=(level < 2))
        width = half
    cand = jnp.concatenate(cur, axis=1)
    colf = jax.lax.broadcasted_iota(jnp.int32, cand.shape, 1).astype(jnp.float32)
    work = cand
    m0r = None
    tr = None
    tops = []
    for it in range(_K):
        m = jnp.max(work, axis=1, keepdims=True)
        if it == 0:
            m0r = m
        tops.append(m)
        cv = jnp.where(work == m, colf, jnp.float32(1e9))
        idx = jnp.min(cv, axis=1, keepdims=True)
        work = jnp.where(cv == idx, -jnp.inf, work)
        tr = m
    t = jnp.maximum(tr, 0.0)
    m0 = jnp.maximum(m0r, 0.0)

    em = jnp.exp(-m0)
    z = (_N - _K) * em
    for v in tops:
        z = z + jnp.exp(jnp.maximum(v, 0.0) - m0)
    ti = jax.lax.bitcast_convert_type(t, jnp.int32)
    t_lo = jax.lax.bitcast_convert_type(ti - 1, jnp.float32)
    thr = jnp.where(t > 0.0, t_lo, 0.0)
    selb = a > thr
    s = m0 + jnp.log(z)
    em_z = em * (1.0 / z)
    out_ref[...] = jnp.where(selb, jnp.exp(a - s), em_z)

    cnt = jnp.zeros((_R, 1), dtype=jnp.float32)
    for j in range(_K):
        cnt = cnt + jnp.sum(
            jnp.where(lanes[j] >= t, 1.0, 0.0), axis=1, keepdims=True
        )
    sat = jnp.max(
        jnp.where(lanes[_K - 1] == t, 1.0, 0.0), axis=1, keepdims=True
    )
    bad = jnp.where(t > 0.0, cnt + 8192.0 * sat, 0.0)
    anomaly = jnp.max(bad) > 10.0

    @pl.when(anomaly)
    def _exact_topk():
        ar = jnp.maximum(a, 0.0)
        colf2 = jax.lax.broadcasted_iota(jnp.int32, ar.shape, 1).astype(jnp.float32)
        wk = ar
        for _ in range(_K):
            mm = jnp.max(wk, axis=1, keepdims=True)
            cv2 = jnp.where(wk == mm, colf2, jnp.float32(1e9))
            ix = jnp.min(cv2, axis=1, keepdims=True)
            wk = jnp.where(cv2 == ix, jnp.float32(-1.0), wk)
        p2 = jnp.where(wk < 0.0, jnp.exp(ar - m0), em)
        z2 = jnp.sum(p2, axis=1, keepdims=True)
        out_ref[...] = p2 * (1.0 / z2)


@jax.jit
def kernel(node_emb):
    et = node_emb.T
    return pl.pallas_call(
        _tc_body,
        grid=(_N // _R,),
        in_specs=[
            pl.BlockSpec((_R, _D), lambda i: (i, 0)),
            pl.BlockSpec((_D, _N), lambda i: (0, 0)),
        ],
        out_specs=pl.BlockSpec((_R, _N), lambda i: (i, 0)),
        out_shape=jax.ShapeDtypeStruct((_N, _N), jnp.float32),
        compiler_params=pltpu.CompilerParams(
            dimension_semantics=("arbitrary",)
        ),
    )(node_emb, et)


# speedup vs baseline: 1.2010x; 1.0007x over previous
import jax
import jax.numpy as jnp
from jax.experimental import pallas as pl
from jax.experimental.pallas import tpu as pltpu

_N = 8192
_D = 16
_K = 10
_R = 256
_W = 128
_C = _N // _W

_SORT10 = [
    (0, 5), (1, 6), (2, 7), (3, 8), (4, 9),
    (0, 3), (1, 4), (5, 8), (6, 9),
    (0, 2), (3, 6), (7, 9),
    (0, 1), (2, 4), (5, 7), (8, 9),
    (1, 2), (3, 5), (4, 6), (7, 8),
    (1, 3), (2, 5), (4, 7), (6, 8),
    (2, 3), (4, 5), (6, 7),
    (3, 4), (5, 6),
]
_SORT4 = [(0, 2), (1, 3), (0, 1), (2, 3), (1, 2)]
_CLEAN = [
    (0, 8), (1, 9), (2, 6), (3, 7), (4, 8), (5, 9),
    (2, 4), (3, 5), (6, 8), (7, 9),
    (0, 1), (2, 3), (4, 5), (6, 7), (8, 9),
]


def _apply_net(vs, net):
    vs = list(vs)
    for i, j in net:
        hi = jnp.maximum(vs[i], vs[j])
        lo = jnp.minimum(vs[i], vs[j])
        vs[i], vs[j] = hi, lo
    return vs


def _merge10(a, b, clean):
    c = [jnp.maximum(a[i], b[_K - 1 - i]) for i in range(_K)]
    if clean:
        c = _apply_net(c, _CLEAN)
    return c


def _tc_body(e_blk_ref, et_ref, out_ref):
    a = jnp.dot(e_blk_ref[...], et_ref[...], preferred_element_type=jnp.float32)

    groups = []
    for g in range(6):
        chunks = [a[:, (10 * g + c) * _W:(10 * g + c + 1) * _W] for c in range(_K)]
        groups.append(_apply_net(chunks, _SORT10))
    rest = [a[:, (60 + c) * _W:(61 + c) * _W] for c in range(4)]
    rest = _apply_net(rest, _SORT4)
    ninf = jnp.full((_R, _W), -jnp.inf, dtype=jnp.float32)
    groups.append(rest + [ninf] * 6)
    m01 = _merge10(groups[0], groups[1], True)
    m23 = _merge10(groups[2], groups[3], True)
    m45 = _merge10(groups[4], groups[5], True)
    ma = _merge10(m01, m23, True)
    mb = _merge10(m45, groups[6], True)
    lanes = _merge10(ma, mb, True)

    cur = lanes
    width = _W
    for level in range(3):
        half = width // 2
        av = [x[:, :half] for x in cur]
        bv = [x[:, half:] for x in cur]
        cur = _merge10(av, bv, clean=(level < 2))
        width = half
    cand = jnp.concatenate(cur, axis=1)
    colf = jax.lax.broadcasted_iota(jnp.int32, cand.shape, 1).astype(jnp.float32)
    work = cand
    m0r = None
    tr = None
    tops = []
    for it in range(_K):
        m = jnp.max(work, axis=1, keepdims=True)
        if it == 0:
            m0r = m
        tops.append(m)
        cv = jnp.where(work == m, colf, jnp.float32(1e9))
        idx = jnp.min(cv, axis=1, keepdims=True)
        work = jnp.where(cv == idx, -jnp.inf, work)
        tr = m
    t = jnp.maximum(tr, 0.0)
    m0 = jnp.maximum(m0r, 0.0)

    em = jnp.exp(-m0)
    z = (_N - _K) * em
    for v in tops:
        z = z + jnp.exp(jnp.maximum(v, 0.0) - m0)
    ti = jax.lax.bitcast_convert_type(t, jnp.int32)
    t_lo = jax.lax.bitcast_convert_type(ti - 1, jnp.float32)
    thr = jnp.where(t > 0.0, t_lo, 0.0)
    selb = a > thr
    s = m0 + jnp.log(z)
    em_z = em * (1.0 / z)
    out_ref[...] = jnp.where(selb, jnp.exp(a - s), em_z)

    cnt = jnp.zeros((_R, 1), dtype=jnp.float32)
    for j in range(_K):
        cnt = cnt + jnp.sum(
            jnp.where(lanes[j] >= t, 1.0, 0.0), axis=1, keepdims=True
        )
    sat = jnp.max(
        jnp.where(lanes[_K - 1] == t, 1.0, 0.0), axis=1, keepdims=True
    )
    bad = jnp.where(t > 0.0, cnt + 8192.0 * sat, 0.0)
    anomaly = jnp.max(bad) > 10.0

    @pl.when(anomaly)
    def _exact_topk():
        ar = jnp.maximum(a, 0.0)
        colf2 = jax.lax.broadcasted_iota(jnp.int32, ar.shape, 1).astype(jnp.float32)
        wk = ar
        for _ in range(_K):
            mm = jnp.max(wk, axis=1, keepdims=True)
            cv2 = jnp.where(wk == mm, colf2, jnp.float32(1e9))
            ix = jnp.min(cv2, axis=1, keepdims=True)
            wk = jnp.where(cv2 == ix, jnp.float32(-1.0), wk)
        p2 = jnp.where(wk < 0.0, jnp.exp(ar - m0), em)
        z2 = jnp.sum(p2, axis=1, keepdims=True)
        out_ref[...] = p2 * (1.0 / z2)


@jax.jit
def kernel(node_emb):
    et = node_emb.T
    return pl.pallas_call(
        _tc_body,
        grid=(_N // _R,),
        in_specs=[
            pl.BlockSpec((_R, _D), lambda i: (i, 0)),
            pl.BlockSpec((_D, _N), lambda i: (0, 0)),
        ],
        out_specs=pl.BlockSpec((_R, _N), lambda i: (i, 0)),
        out_shape=jax.ShapeDtypeStruct((_N, _N), jnp.float32),
        compiler_params=pltpu.CompilerParams(
            dimension_semantics=("parallel",)
        ),
    )(node_emb, et)


# speedup vs baseline: 1.3550x; 1.1282x over previous
import jax
import jax.numpy as jnp
from jax.experimental import pallas as pl
from jax.experimental.pallas import tpu as pltpu

_N = 8192
_D = 16
_K = 10
_R = 256
_W = 128
_C = _N // _W

_SORT5 = [(0, 3), (1, 4), (0, 2), (1, 3), (0, 1), (2, 4), (1, 2), (3, 4), (2, 3)]
_SORT4 = [(0, 2), (1, 3), (0, 1), (2, 3), (1, 2)]
_M55TOP5 = [(0, 4), (1, 3), (2, 4), (1, 2), (3, 4)]
_CLEAN = [
    (0, 8), (1, 9), (2, 6), (3, 7), (4, 8), (5, 9),
    (2, 4), (3, 5), (6, 8), (7, 9),
    (0, 1), (2, 3), (4, 5), (6, 7), (8, 9),
]


def _apply_net(vs, net):
    vs = list(vs)
    for i, j in net:
        hi = jnp.maximum(vs[i], vs[j])
        lo = jnp.minimum(vs[i], vs[j])
        vs[i], vs[j] = hi, lo
    return vs


def _merge10(a, b, clean):
    c = [jnp.maximum(a[i], b[_K - 1 - i]) for i in range(_K)]
    if clean:
        c = _apply_net(c, _CLEAN)
    return c


def _merge5(a, b):
    c = [jnp.maximum(a[i], b[4 - i]) for i in range(5)]
    return _apply_net(c, _M55TOP5)


def _tc_body(e_blk_ref, et_ref, out_ref):
    a = jnp.dot(e_blk_ref[...], et_ref[...], preferred_element_type=jnp.float32)

    lists5 = []
    for g in range(12):
        chunks = [a[:, (5 * g + c) * _W:(5 * g + c + 1) * _W] for c in range(5)]
        lists5.append(_apply_net(chunks, _SORT5))
    rest = [a[:, (60 + c) * _W:(61 + c) * _W] for c in range(4)]
    rest = _apply_net(rest, _SORT4)
    ninf = jnp.full((_R, _W), -jnp.inf, dtype=jnp.float32)
    lists5.append(rest + [ninf])
    cur5 = lists5
    while len(cur5) > 1:
        nxt = [_merge5(cur5[i], cur5[i + 1]) for i in range(0, len(cur5) - 1, 2)]
        if len(cur5) % 2:
            nxt.append(cur5[-1])
        cur5 = nxt
    lanes = cur5[0]

    av = [x[:, :64] for x in lanes]
    bv = [x[:, 64:] for x in lanes]
    l10 = _apply_net(av + bv[::-1], _CLEAN)
    av = [x[:, :32] for x in l10]
    bv = [x[:, 32:] for x in l10]
    l10 = _merge10(av, bv, True)
    av = [x[:, :16] for x in l10]
    bv = [x[:, 16:] for x in l10]
    cur = [jnp.maximum(av[k], bv[_K - 1 - k]) for k in range(_K)]
    cand = jnp.concatenate(cur, axis=1)
    colf = jax.lax.broadcasted_iota(jnp.int32, cand.shape, 1).astype(jnp.float32)
    work = cand
    m0r = None
    tr = None
    tops = []
    for it in range(_K):
        m = jnp.max(work, axis=1, keepdims=True)
        if it == 0:
            m0r = m
        tops.append(m)
        cv = jnp.where(work == m, colf, jnp.float32(1e9))
        idx = jnp.min(cv, axis=1, keepdims=True)
        work = jnp.where(cv == idx, -jnp.inf, work)
        tr = m
    t = jnp.maximum(tr, 0.0)
    m0 = jnp.maximum(m0r, 0.0)

    em = jnp.exp(-m0)
    z = (_N - _K) * em
    for v in tops:
        z = z + jnp.exp(jnp.maximum(v, 0.0) - m0)
    ti = jax.lax.bitcast_convert_type(t, jnp.int32)
    t_lo = jax.lax.bitcast_convert_type(ti - 1, jnp.float32)
    thr = jnp.where(t > 0.0, t_lo, 0.0)
    selb = a > thr
    s = m0 + jnp.log(z)
    em_z = em * (1.0 / z)
    out_ref[...] = jnp.where(selb, jnp.exp(a - s), em_z)

    trig = jnp.max(
        jnp.where(lanes[4] >= t, 1.0, 0.0), axis=1, keepdims=True
    )
    cnt = jnp.zeros((_R, 1), dtype=jnp.float32)
    for j in range(5):
        cnt = cnt + jnp.sum(
            jnp.where(lanes[j] >= t, 1.0, 0.0), axis=1, keepdims=True
        )
    bad = jnp.where(t > 0.0, cnt, 0.0) + 8192.0 * trig
    anomaly = jnp.max(bad) > 10.0

    @pl.when(anomaly)
    def _exact_topk():
        ar = jnp.maximum(a, 0.0)
        colf2 = jax.lax.broadcasted_iota(jnp.int32, ar.shape, 1).astype(jnp.float32)
        wk = ar
        for _ in range(_K):
            mm = jnp.max(wk, axis=1, keepdims=True)
            cv2 = jnp.where(wk == mm, colf2, jnp.float32(1e9))
            ix = jnp.min(cv2, axis=1, keepdims=True)
            wk = jnp.where(cv2 == ix, jnp.float32(-1.0), wk)
        p2 = jnp.where(wk < 0.0, jnp.exp(ar - m0), em)
        z2 = jnp.sum(p2, axis=1, keepdims=True)
        out_ref[...] = p2 * (1.0 / z2)


@jax.jit
def kernel(node_emb):
    et = node_emb.T
    return pl.pallas_call(
        _tc_body,
        grid=(_N // _R,),
        in_specs=[
            pl.BlockSpec((_R, _D), lambda i: (i, 0)),
            pl.BlockSpec((_D, _N), lambda i: (0, 0)),
        ],
        out_specs=pl.BlockSpec((_R, _N), lambda i: (i, 0)),
        out_shape=jax.ShapeDtypeStruct((_N, _N), jnp.float32),
        compiler_params=pltpu.CompilerParams(
            dimension_semantics=("parallel",)
        ),
    )(node_emb, et)
